# baseline (speedup 1.0000x reference)
import jax
import jax.numpy as jnp
from jax.experimental import pallas as pl

N_U = 50000
N_I = 50000
D = 64
B = 4096


def _predict_body(zu_pos_ref, zu_neg_ref, zi_pos_ref, zi_neg_ref, p1_ref, p2_ref,
                  scores_ref):
    a = jnp.tanh(zu_pos_ref[...] @ p1_ref[...])
    b = jnp.tanh(zi_pos_ref[...] @ p2_ref[...])
    c = jnp.tanh(zi_neg_ref[...] @ p2_ref[...])
    d = jnp.tanh(zu_neg_ref[...] @ p1_ref[...])
    pos = jnp.sum(a * b, axis=-1, keepdims=True)
    neg_u = jnp.sum(a * c, axis=-1, keepdims=True)
    neg_i = jnp.sum(d * b, axis=-1, keepdims=True)
    scores_ref[...] = jnp.concatenate([pos, neg_u, neg_i], axis=-1)


def _predict_scores(zu_pos, zu_neg, zi_pos, zi_neg, P1, P2):
    return pl.pallas_call(
        _predict_body,
        out_shape=jax.ShapeDtypeStruct((B, 3), jnp.float32),
    )(zu_pos, zu_neg, zi_pos, zi_neg, P1, P2)


def _evolve(adj, t_diff, xu_in, xi_in, eu, ei, Wu, Wi, w_decay):
    decay = jnp.exp(-jax.nn.softplus(w_decay) * t_diff)
    u_idx = adj[0]
    i_idx = adj[1]
    ones = jnp.ones((adj.shape[1],), dtype=xu_in.dtype)
    msg_u = jnp.take(xi_in, i_idx, axis=0)
    agg_u = jax.ops.segment_sum(msg_u, u_idx, num_segments=N_U)
    deg_u = jax.ops.segment_sum(ones, u_idx, num_segments=N_U)
    agg_u = agg_u / jnp.clip(deg_u, 1.0, None)[:, None]
    msg_i = jnp.take(xu_in, u_idx, axis=0)
    agg_i = jax.ops.segment_sum(msg_i, i_idx, num_segments=N_I)
    deg_i = jax.ops.segment_sum(ones, i_idx, num_segments=N_I)
    agg_i = agg_i / jnp.clip(deg_i, 1.0, None)[:, None]
    xu_out = decay * xu_in + jnp.tanh(agg_u @ Wu + eu)
    xi_out = decay * xi_in + jnp.tanh(agg_i @ Wi + ei)
    return xu_out, xi_out


def _update(xu, xi, adj_i2u, adj_u2i, Uu, Ui):
    m_u = jnp.tanh(jnp.take(xi, adj_i2u[0], axis=0) @ Uu)
    dxu = jax.ops.segment_sum(m_u, adj_i2u[1], num_segments=N_U)
    m_i = jnp.tanh(jnp.take(xu, adj_u2i[0], axis=0) @ Ui)
    dxi = jax.ops.segment_sum(m_i, adj_u2i[1], num_segments=N_I)
    return dxu, dxi


def kernel(t_diff, xu_in_his, xi_in_his, xu_in_ctx, xi_in_ctx, embeds_u, embeds_i, Wu_his, Wi_his, w_decay_his, Wu_ctx, Wi_ctx, w_decay_ctx, P1, P2, Uu_his, Ui_his, Uu_ctx, Ui_ctx, adj_his, adj_ctx, adj_tgt_i2u, adj_tgt_u2i, tgt_u, tgt_i, tgt_u_neg, tgt_i_neg):
    xu_tm_his, xi_tm_his = _evolve(adj_his, t_diff, xu_in_his, xi_in_his, embeds_u, embeds_i, Wu_his, Wi_his, w_decay_his)
    xu_tm_ctx, xi_tm_ctx = _evolve(adj_ctx, t_diff, xu_in_ctx, xi_in_ctx, embeds_u, embeds_i, Wu_ctx, Wi_ctx, w_decay_ctx)
    zu_t = xu_tm_his + xu_tm_ctx
    zi_t = xi_tm_his + xi_tm_ctx
    zu_enc = jnp.concatenate([zu_t, embeds_u], axis=1)
    zi_enc = jnp.concatenate([zi_t, embeds_i], axis=1)
    zu_pos = jnp.take(zu_enc, tgt_u, axis=0)
    zu_neg = jnp.take(zu_enc, tgt_u_neg, axis=0)
    zi_pos = jnp.take(zi_enc, tgt_i, axis=0)
    zi_neg = jnp.take(zi_enc, tgt_i_neg, axis=0)
    scores = _predict_scores(zu_pos, zu_neg, zi_pos, zi_neg, P1, P2)
    loss = -jnp.mean(jax.nn.log_softmax(scores, axis=1)[:, 0])
    dxu_his, dxi_his = _update(xu_tm_his, xi_tm_his, adj_tgt_i2u, adj_tgt_u2i, Uu_his, Ui_his)
    dxu_ctx, dxi_ctx = _update(xu_tm_ctx, xi_tm_ctx, adj_tgt_i2u, adj_tgt_u2i, Uu_ctx, Ui_ctx)
    return (loss, zu_pos, zi_enc, xu_tm_his + dxu_his, xi_tm_his + dxi_his, xu_tm_ctx + dxu_ctx, xi_tm_ctx + dxi_ctx)
